# baseline (speedup 1.0000x reference)
import jax
import jax.numpy as jnp
from jax import lax
from jax.experimental import pallas as pl
from jax.experimental.pallas import tpu as pltpu, tpu_sc as plsc

N, E, D, H, O = 10000, 320000, 128, 128, 1
NC, NS = 2, 16
NW = NC * NS
EPW = E // NW
CW = 125
CH = EPW // CW
EPS = E // NS
CH2 = EPS // CW
HH = H // NC
NP = 10240
RPS = NP // NS
BM = 2048


def _mish(v):
    sp = jnp.maximum(v, 0.0) + jnp.log1p(jnp.exp(-jnp.abs(v)))
    return v * jnp.tanh(sp)


_MESH = plsc.VectorSubcoreMesh(core_axis_name="c", subcore_axis_name="s")


def _deg_body(dst_hbm, out_hbm, idx_d, ones_v, zbuf, acc):
    c = lax.axis_index("c")
    s = lax.axis_index("s")
    wid = c * NS + s

    zeros = jnp.zeros((16,), jnp.float32)
    ones = jnp.ones((16,), jnp.float32)

    def zb(k, carry):
        zbuf[k, pl.ds(0, 16)] = zeros
        return carry

    lax.fori_loop(0, 128, zb, 0)
    for q in range(RPS // 128):
        pltpu.sync_copy(zbuf, acc.at[pl.ds(s * RPS + q * 128, 128)])

    def ob(k, carry):
        ones_v[k, pl.ds(0, 16)] = ones
        return carry

    lax.fori_loop(0, CW, ob, 0)
    plsc.subcore_barrier()

    pltpu.sync_copy(dst_hbm.at[wid], idx_d)

    def body(j, carry):
        pltpu.sync_copy(ones_v, acc.at[idx_d.at[j]], add=True)
        return carry

    lax.fori_loop(0, CH, body, 0)
    plsc.subcore_barrier()
    pltpu.sync_copy(acc.at[pl.ds(s * RPS, RPS)], out_hbm.at[c, pl.ds(s * RPS, RPS)])


_deg_call = pl.kernel(
    _deg_body,
    out_type=jax.ShapeDtypeStruct((NC, NP, 16), jnp.float32),
    mesh=_MESH,
    compiler_params=pltpu.CompilerParams(use_tc_tiling_on_sc=False),
    scratch_types=[
        pltpu.VMEM((CH, CW), jnp.int32),
        pltpu.VMEM((CW, 16), jnp.float32),
        pltpu.VMEM((128, 16), jnp.float32),
        pltpu.VMEM_SHARED((NP, 16), jnp.float32),
    ],
)


_NBUF = 4


def _agg_body(t_hbm, src_hbm, dst_hbm, out_hbm, idx_s, idx_d, rows, zbuf,
              acc, sems):
    c = lax.axis_index("c")
    s = lax.axis_index("s")

    pltpu.sync_copy(src_hbm.at[s], idx_s)
    pltpu.sync_copy(dst_hbm.at[s], idx_d)
    tbl = t_hbm.at[c]

    for b in range(_NBUF):
        pltpu.async_copy(tbl.at[idx_s.at[b]], rows[b], sems[b])

    zeros = jnp.zeros((16,), jnp.float32)

    def zb(k, carry):
        zbuf[k // 4, pl.ds((k % 4) * 16, 16)] = zeros
        return carry

    lax.fori_loop(0, 128 * (HH // 16), zb, 0)
    for q in range(RPS // 128):
        pltpu.sync_copy(zbuf, acc.at[pl.ds(s * RPS + q * 128, 128)])
    plsc.subcore_barrier()

    def body(jj, carry):
        j = _NBUF * jj
        for b in range(_NBUF):
            jb = j + b
            pltpu.make_async_copy(tbl.at[idx_s.at[jb]], rows[b], sems[b]).wait()
            pltpu.sync_copy(rows[b], acc.at[idx_d.at[jb]], add=True)

            @pl.when(jb + _NBUF < CH2)
            def _next():
                pltpu.async_copy(tbl.at[idx_s.at[jb + _NBUF]], rows[b], sems[b])

        return carry

    lax.fori_loop(0, CH2 // _NBUF, body, 0)
    plsc.subcore_barrier()
    pltpu.sync_copy(acc.at[pl.ds(s * RPS, RPS)], out_hbm.at[c, pl.ds(s * RPS, RPS)])


_agg_call = pl.kernel(
    _agg_body,
    out_type=jax.ShapeDtypeStruct((NC, NP, HH), jnp.float32),
    mesh=_MESH,
    compiler_params=pltpu.CompilerParams(use_tc_tiling_on_sc=False),
    scratch_types=[
        pltpu.VMEM((CH2, CW), jnp.int32),
        pltpu.VMEM((CH2, CW), jnp.int32),
        [pltpu.VMEM((CW, HH), jnp.float32) for _ in range(_NBUF)],
        pltpu.VMEM((128, HH), jnp.float32),
        pltpu.VMEM_SHARED((NP, HH), jnp.float32),
        [pltpu.SemaphoreType.DMA for _ in range(_NBUF)],
    ],
)


def _pre_body(x_ref, w_ref, b_ref, degp_ref, wg_ref, t_ref, h_ref, dinv_ref):
    h = _mish(
        jnp.dot(x_ref[...], w_ref[...], preferred_element_type=jnp.float32)
        + b_ref[...]
    )
    deg = degp_ref[0, :, 0] + degp_ref[1, :, 0] + 1.0
    dinv = lax.rsqrt(deg)[:, None]
    dinv_ref[...] = dinv
    h_ref[...] = h
    t = dinv * jnp.dot(h, wg_ref[...], preferred_element_type=jnp.float32)
    t_ref[0] = t[:, :HH]
    t_ref[1] = t[:, HH:]


def _tc_pre(x, w, b, degp, wg):
    return pl.pallas_call(
        _pre_body,
        grid=(pl.cdiv(N, BM),),
        in_specs=[
            pl.BlockSpec((BM, D), lambda i: (i, 0)),
            pl.BlockSpec((D, H), lambda i: (0, 0)),
            pl.BlockSpec((1, H), lambda i: (0, 0)),
            pl.BlockSpec((NC, BM, 16), lambda i: (0, i, 0)),
            pl.BlockSpec((H, H), lambda i: (0, 0)),
        ],
        out_specs=[
            pl.BlockSpec((NC, BM, HH), lambda i: (0, i, 0)),
            pl.BlockSpec((BM, H), lambda i: (i, 0)),
            pl.BlockSpec((BM, 1), lambda i: (i, 0)),
        ],
        out_shape=[
            jax.ShapeDtypeStruct((NC, N, HH), jnp.float32),
            jax.ShapeDtypeStruct((N, H), jnp.float32),
            jax.ShapeDtypeStruct((N, 1), jnp.float32),
        ],
    )(x, w, b, degp, wg)


def _u_body(h_ref, t_ref, dinv_ref, wl_ref, bl_ref, u_ref):
    tfull = jnp.concatenate([t_ref[0], t_ref[1]], axis=-1)
    u_ref[...] = (
        jnp.dot(h_ref[...], wl_ref[...], preferred_element_type=jnp.float32)
        + bl_ref[...]
        + dinv_ref[...] * tfull
    )


def _tc_u(h, t, dinv, wl, bl):
    return pl.pallas_call(
        _u_body,
        grid=(pl.cdiv(N, BM),),
        in_specs=[
            pl.BlockSpec((BM, H), lambda i: (i, 0)),
            pl.BlockSpec((NC, BM, HH), lambda i: (0, i, 0)),
            pl.BlockSpec((BM, 1), lambda i: (i, 0)),
            pl.BlockSpec((H, H), lambda i: (0, 0)),
            pl.BlockSpec((1, H), lambda i: (0, 0)),
        ],
        out_specs=pl.BlockSpec((BM, H), lambda i: (i, 0)),
        out_shape=jax.ShapeDtypeStruct((N, H), jnp.float32),
    )(h, t, dinv, wl, bl)


def _combine(a_ref, u_ref, dinv_ref, bg_ref):
    agg = jnp.concatenate([a_ref[0], a_ref[1]], axis=-1)
    return _mish(dinv_ref[...] * agg + bg_ref[...] + u_ref[...])


def _comb_mm_body(a_ref, u_ref, dinv_ref, bg_ref, wg_ref, t2_ref, h2_ref):
    h = _combine(a_ref, u_ref, dinv_ref, bg_ref)
    h2_ref[...] = h
    t = dinv_ref[...] * jnp.dot(h, wg_ref[...], preferred_element_type=jnp.float32)
    t2_ref[0] = t[:, :HH]
    t2_ref[1] = t[:, HH:]


def _tc_comb_mm(a, u, dinv, bg, wg):
    return pl.pallas_call(
        _comb_mm_body,
        grid=(pl.cdiv(N, BM),),
        in_specs=[
            pl.BlockSpec((NC, BM, HH), lambda i: (0, i, 0)),
            pl.BlockSpec((BM, H), lambda i: (i, 0)),
            pl.BlockSpec((BM, 1), lambda i: (i, 0)),
            pl.BlockSpec((1, H), lambda i: (0, 0)),
            pl.BlockSpec((H, H), lambda i: (0, 0)),
        ],
        out_specs=[
            pl.BlockSpec((NC, BM, HH), lambda i: (0, i, 0)),
            pl.BlockSpec((BM, H), lambda i: (i, 0)),
        ],
        out_shape=[
            jax.ShapeDtypeStruct((NC, N, HH), jnp.float32),
            jax.ShapeDtypeStruct((N, H), jnp.float32),
        ],
    )(a, u, dinv, bg, wg)


def _comb_post_body(a_ref, u_ref, dinv_ref, bg_ref, wp_ref, bp_ref, o_ref):
    h = _combine(a_ref, u_ref, dinv_ref, bg_ref)
    o_ref[...] = (
        jnp.dot(h, wp_ref[...], preferred_element_type=jnp.float32) + bp_ref[...]
    )


def _tc_comb_post(a, u, dinv, bg, wp, bp):
    return pl.pallas_call(
        _comb_post_body,
        grid=(pl.cdiv(N, BM),),
        in_specs=[
            pl.BlockSpec((NC, BM, HH), lambda i: (0, i, 0)),
            pl.BlockSpec((BM, H), lambda i: (i, 0)),
            pl.BlockSpec((BM, 1), lambda i: (i, 0)),
            pl.BlockSpec((1, H), lambda i: (0, 0)),
            pl.BlockSpec((H, O), lambda i: (0, 0)),
            pl.BlockSpec((1, O), lambda i: (0, 0)),
        ],
        out_specs=pl.BlockSpec((BM, O), lambda i: (i, 0)),
        out_shape=jax.ShapeDtypeStruct((N, O), jnp.float32),
    )(a, u, dinv, bg, wp, bp)


def kernel(x, edge_index, W_pre, b_pre, Wg0, bg0, Wl0, bl0, Wg1, bg1, Wl1, bl1,
           Wg2, bg2, Wl2, bl2, W_post, b_post):
    src = edge_index[0].reshape(NS, CH2, CW)
    dst = edge_index[1].reshape(NS, CH2, CW)
    dstd = edge_index[1].reshape(NW, CH, CW)

    degp = _deg_call(dstd)
    T, h, dinv = _tc_pre(x, W_pre, b_pre.reshape(1, H), degp, Wg0)
    A = _agg_call(T, src, dst)
    U = _tc_u(h, T, dinv, Wl0, bl0.reshape(1, H))
    for bg, Wg, Wl, bl in ((bg0, Wg1, Wl1, bl1), (bg1, Wg2, Wl2, bl2)):
        T, h = _tc_comb_mm(A, U, dinv, bg.reshape(1, H), Wg)
        A = _agg_call(T, src, dst)
        U = _tc_u(h, T, dinv, Wl, bl.reshape(1, H))
    return _tc_comb_post(A, U, dinv, bg2.reshape(1, H),
                         W_post, b_post.reshape(1, O))


# speedup vs baseline: 1.0298x; 1.0298x over previous
import jax
import jax.numpy as jnp
from jax import lax
from jax.experimental import pallas as pl
from jax.experimental.pallas import tpu as pltpu, tpu_sc as plsc

N, E, D, H, O = 10000, 320000, 128, 128, 1
NC, NS = 2, 16
NW = NC * NS
EPW = E // NW
CW = 125
CH = EPW // CW
EPS = E // NS
CH2 = EPS // CW
HH = H // NC
NP = 10240
RPS = NP // NS
BM = 2048


def _mish(v):
    sp = jnp.maximum(v, 0.0) + jnp.log1p(jnp.exp(-jnp.abs(v)))
    return v * jnp.tanh(sp)


_MESH = plsc.VectorSubcoreMesh(core_axis_name="c", subcore_axis_name="s")


def _deg_body(dst_hbm, out_hbm, idx_d, ones_v, zbuf, acc, sem):
    c = lax.axis_index("c")
    s = lax.axis_index("s")
    wid = c * NS + s

    zeros = jnp.zeros((16,), jnp.float32)
    ones = jnp.ones((16,), jnp.float32)

    def zb(k, carry):
        zbuf[k, pl.ds(0, 16)] = zeros
        return carry

    lax.fori_loop(0, 128, zb, 0)
    for q in range(RPS // 128):
        pltpu.sync_copy(zbuf, acc.at[pl.ds(s * RPS + q * 128, 128)])

    def ob(k, carry):
        ones_v[k, pl.ds(0, 16)] = ones
        return carry

    lax.fori_loop(0, CW, ob, 0)
    plsc.subcore_barrier()

    pltpu.sync_copy(dst_hbm.at[wid], idx_d)

    def body(j, carry):
        pltpu.async_copy(ones_v, acc.at[idx_d.at[j]], sem, add=True)
        return carry

    lax.fori_loop(0, CH, body, 0)

    def drain(j, carry):
        pltpu.make_async_copy(ones_v, acc.at[idx_d.at[j]], sem).wait()
        return carry

    lax.fori_loop(0, CH, drain, 0)
    plsc.subcore_barrier()
    pltpu.sync_copy(acc.at[pl.ds(s * RPS, RPS)], out_hbm.at[c, pl.ds(s * RPS, RPS)])


_deg_call = pl.kernel(
    _deg_body,
    out_type=jax.ShapeDtypeStruct((NC, NP, 16), jnp.float32),
    mesh=_MESH,
    compiler_params=pltpu.CompilerParams(use_tc_tiling_on_sc=False),
    scratch_types=[
        pltpu.VMEM((CH, CW), jnp.int32),
        pltpu.VMEM((CW, 16), jnp.float32),
        pltpu.VMEM((128, 16), jnp.float32),
        pltpu.VMEM_SHARED((NP, 16), jnp.float32),
        pltpu.SemaphoreType.DMA,
    ],
)


_NBUF = 4


def _agg_body(t_hbm, src_hbm, dst_hbm, out_hbm, idx_s, idx_d, rows, zbuf,
              acc, sems):
    c = lax.axis_index("c")
    s = lax.axis_index("s")

    pltpu.sync_copy(src_hbm.at[s], idx_s)
    pltpu.sync_copy(dst_hbm.at[s], idx_d)
    tbl = t_hbm.at[c]

    for b in range(_NBUF):
        pltpu.async_copy(tbl.at[idx_s.at[b]], rows[b], sems[b])

    zeros = jnp.zeros((16,), jnp.float32)

    def zb(k, carry):
        zbuf[k // 4, pl.ds((k % 4) * 16, 16)] = zeros
        return carry

    lax.fori_loop(0, 128 * (HH // 16), zb, 0)
    for q in range(RPS // 128):
        pltpu.sync_copy(zbuf, acc.at[pl.ds(s * RPS + q * 128, 128)])
    plsc.subcore_barrier()

    def body(jj, carry):
        j = _NBUF * jj
        for b in range(_NBUF):
            jb = j + b
            pltpu.make_async_copy(tbl.at[idx_s.at[jb]], rows[b], sems[b]).wait()
            pltpu.sync_copy(rows[b], acc.at[idx_d.at[jb]], add=True)

            @pl.when(jb + _NBUF < CH2)
            def _next():
                pltpu.async_copy(tbl.at[idx_s.at[jb + _NBUF]], rows[b], sems[b])

        return carry

    lax.fori_loop(0, CH2 // _NBUF, body, 0)
    plsc.subcore_barrier()
    pltpu.sync_copy(acc.at[pl.ds(s * RPS, RPS)], out_hbm.at[c, pl.ds(s * RPS, RPS)])


_agg_call = pl.kernel(
    _agg_body,
    out_type=jax.ShapeDtypeStruct((NC, NP, HH), jnp.float32),
    mesh=_MESH,
    compiler_params=pltpu.CompilerParams(use_tc_tiling_on_sc=False),
    scratch_types=[
        pltpu.VMEM((CH2, CW), jnp.int32),
        pltpu.VMEM((CH2, CW), jnp.int32),
        [pltpu.VMEM((CW, HH), jnp.float32) for _ in range(_NBUF)],
        pltpu.VMEM((128, HH), jnp.float32),
        pltpu.VMEM_SHARED((NP, HH), jnp.float32),
        [pltpu.SemaphoreType.DMA for _ in range(_NBUF)],
    ],
)


def _dual_mm(h, dinv, wg_ref, wl_ref, bl_ref, t_ref, u_ref):
    t = dinv * jnp.dot(h, wg_ref[...], preferred_element_type=jnp.float32)
    t_ref[0] = t[:, :HH]
    t_ref[1] = t[:, HH:]
    u_ref[...] = (
        jnp.dot(h, wl_ref[...], preferred_element_type=jnp.float32)
        + bl_ref[...]
        + dinv * t
    )


def _pre_body(x_ref, w_ref, b_ref, degp_ref, wg_ref, wl_ref, bl_ref,
              t_ref, u_ref, dinv_ref):
    h = _mish(
        jnp.dot(x_ref[...], w_ref[...], preferred_element_type=jnp.float32)
        + b_ref[...]
    )
    deg = degp_ref[0, :, 0] + degp_ref[1, :, 0] + 1.0
    dinv = lax.rsqrt(deg)[:, None]
    dinv_ref[...] = dinv
    _dual_mm(h, dinv, wg_ref, wl_ref, bl_ref, t_ref, u_ref)


def _tc_pre(x, w, b, degp, wg, wl, bl):
    return pl.pallas_call(
        _pre_body,
        grid=(pl.cdiv(N, BM),),
        in_specs=[
            pl.BlockSpec((BM, D), lambda i: (i, 0)),
            pl.BlockSpec((D, H), lambda i: (0, 0)),
            pl.BlockSpec((1, H), lambda i: (0, 0)),
            pl.BlockSpec((NC, BM, 16), lambda i: (0, i, 0)),
            pl.BlockSpec((H, H), lambda i: (0, 0)),
            pl.BlockSpec((H, H), lambda i: (0, 0)),
            pl.BlockSpec((1, H), lambda i: (0, 0)),
        ],
        out_specs=[
            pl.BlockSpec((NC, BM, HH), lambda i: (0, i, 0)),
            pl.BlockSpec((BM, H), lambda i: (i, 0)),
            pl.BlockSpec((BM, 1), lambda i: (i, 0)),
        ],
        out_shape=[
            jax.ShapeDtypeStruct((NC, N, HH), jnp.float32),
            jax.ShapeDtypeStruct((N, H), jnp.float32),
            jax.ShapeDtypeStruct((N, 1), jnp.float32),
        ],
    )(x, w, b, degp, wg, wl, bl)


def _combine(a_ref, u_ref, dinv_ref, bg_ref):
    agg = jnp.concatenate([a_ref[0], a_ref[1]], axis=-1)
    return _mish(dinv_ref[...] * agg + bg_ref[...] + u_ref[...])


def _comb_mm_body(a_ref, u_ref, dinv_ref, bg_ref, wg_ref, wl_ref, bl_ref,
                  t2_ref, u2_ref):
    h = _combine(a_ref, u_ref, dinv_ref, bg_ref)
    _dual_mm(h, dinv_ref[...], wg_ref, wl_ref, bl_ref, t2_ref, u2_ref)


def _tc_comb_mm(a, u, dinv, bg, wg, wl, bl):
    return pl.pallas_call(
        _comb_mm_body,
        grid=(pl.cdiv(N, BM),),
        in_specs=[
            pl.BlockSpec((NC, BM, HH), lambda i: (0, i, 0)),
            pl.BlockSpec((BM, H), lambda i: (i, 0)),
            pl.BlockSpec((BM, 1), lambda i: (i, 0)),
            pl.BlockSpec((1, H), lambda i: (0, 0)),
            pl.BlockSpec((H, H), lambda i: (0, 0)),
            pl.BlockSpec((H, H), lambda i: (0, 0)),
            pl.BlockSpec((1, H), lambda i: (0, 0)),
        ],
        out_specs=[
            pl.BlockSpec((NC, BM, HH), lambda i: (0, i, 0)),
            pl.BlockSpec((BM, H), lambda i: (i, 0)),
        ],
        out_shape=[
            jax.ShapeDtypeStruct((NC, N, HH), jnp.float32),
            jax.ShapeDtypeStruct((N, H), jnp.float32),
        ],
    )(a, u, dinv, bg, wg, wl, bl)


def _comb_post_body(a_ref, u_ref, dinv_ref, bg_ref, wp_ref, bp_ref, o_ref):
    h = _combine(a_ref, u_ref, dinv_ref, bg_ref)
    o_ref[...] = (
        jnp.dot(h, wp_ref[...], preferred_element_type=jnp.float32) + bp_ref[...]
    )


def _tc_comb_post(a, u, dinv, bg, wp, bp):
    return pl.pallas_call(
        _comb_post_body,
        grid=(pl.cdiv(N, BM),),
        in_specs=[
            pl.BlockSpec((NC, BM, HH), lambda i: (0, i, 0)),
            pl.BlockSpec((BM, H), lambda i: (i, 0)),
            pl.BlockSpec((BM, 1), lambda i: (i, 0)),
            pl.BlockSpec((1, H), lambda i: (0, 0)),
            pl.BlockSpec((H, O), lambda i: (0, 0)),
            pl.BlockSpec((1, O), lambda i: (0, 0)),
        ],
        out_specs=pl.BlockSpec((BM, O), lambda i: (i, 0)),
        out_shape=jax.ShapeDtypeStruct((N, O), jnp.float32),
    )(a, u, dinv, bg, wp, bp)


def kernel(x, edge_index, W_pre, b_pre, Wg0, bg0, Wl0, bl0, Wg1, bg1, Wl1, bl1,
           Wg2, bg2, Wl2, bl2, W_post, b_post):
    src = edge_index[0].reshape(NS, CH2, CW)
    dst = edge_index[1].reshape(NS, CH2, CW)
    dstd = edge_index[1].reshape(NW, CH, CW)

    degp = _deg_call(dstd)
    T, U, dinv = _tc_pre(x, W_pre, b_pre.reshape(1, H), degp,
                         Wg0, Wl0, bl0.reshape(1, H))
    A = _agg_call(T, src, dst)
    for bg, Wg, Wl, bl in ((bg0, Wg1, Wl1, bl1), (bg1, Wg2, Wl2, bl2)):
        T, U = _tc_comb_mm(A, U, dinv, bg.reshape(1, H),
                           Wg, Wl, bl.reshape(1, H))
        A = _agg_call(T, src, dst)
    return _tc_comb_post(A, U, dinv, bg2.reshape(1, H),
                         W_post, b_post.reshape(1, O))
